# baseline (speedup 1.0000x reference)
import functools

import jax
import jax.numpy as jnp
from jax import lax
from jax.experimental import pallas as pl
from jax.experimental.pallas import tpu as pltpu
from jax.experimental.pallas import tpu_sc as plsc

NC, NS, LANES = 2, 16, 16
NW = NC * NS
CH = 1024
CHB = 256
BLK = 2048


def _cdiv(a, b):
    return (a + b - 1) // b


def _mesh():
    return plsc.VectorSubcoreMesh(
        core_axis_name="c", subcore_axis_name="s",
        num_cores=NC, num_subcores=NS)


def _tc1(x_p, w1, as1, ad1, np_, blk):
    def body(x_ref, w1_ref, as_ref, ad_ref, ts_ref, td_ref):
        h = jnp.dot(x_ref[...], w1_ref[...],
                    preferred_element_type=jnp.float32)
        ri = lax.broadcasted_iota(jnp.int32, (64, 8), 0)
        ci = lax.broadcasted_iota(jnp.int32, (64, 8), 1)
        msk = (ci == (ri >> 3)).astype(jnp.float32)
        ts_ref[...] = jnp.dot(h, as_ref[...] * msk,
                              preferred_element_type=jnp.float32)
        td_ref[...] = jnp.dot(h, ad_ref[...] * msk,
                              preferred_element_type=jnp.float32)

    grid = (np_ // blk,)
    return pl.pallas_call(
        body,
        grid=grid,
        in_specs=[
            pl.BlockSpec((blk, 2), lambda i: (i, 0)),
            pl.BlockSpec((2, 64), lambda i: (0, 0)),
            pl.BlockSpec((64, 1), lambda i: (0, 0)),
            pl.BlockSpec((64, 1), lambda i: (0, 0)),
        ],
        out_specs=[
            pl.BlockSpec((blk, 8), lambda i: (i, 0)),
            pl.BlockSpec((blk, 8), lambda i: (i, 0)),
        ],
        out_shape=[
            jax.ShapeDtypeStruct((np_, 8), jnp.float32),
            jax.ShapeDtypeStruct((np_, 8), jnp.float32),
        ],
    )(x_p, w1, as1, ad1)


def _tc2(den_p, n, np_, blk):
    def body(dp_ref, rd_ref):
        i = pl.program_id(0)
        d = dp_ref[0] + dp_ref[1]
        rows = i * blk + lax.broadcasted_iota(jnp.int32, (blk, 8), 0)
        rd_ref[...] = jnp.where(rows < n, 1.0 / (d + 1e-16), 0.0)

    return pl.pallas_call(
        body,
        grid=(np_ // blk,),
        in_specs=[pl.BlockSpec((2, blk, 8), lambda i: (0, i, 0))],
        out_specs=pl.BlockSpec((blk, 8), lambda i: (i, 0)),
        out_shape=jax.ShapeDtypeStruct((np_, 8), jnp.float32),
    )(den_p)


def _tc3(g_p, rden, w1, b1r, w2, as2, ad2, np_, blk):
    def body(gp_ref, rd_ref, w1_ref, b1_ref, w2_ref, as2_ref, ad2_ref,
             t2_ref):
        g = gp_ref[0] + gp_ref[1]
        rd = rd_ref[...]
        gn0 = g[:, 0:8] * rd
        gn1 = g[:, 8:16] * rd
        ri = lax.broadcasted_iota(jnp.int32, (8, 64), 0)
        ci = lax.broadcasted_iota(jnp.int32, (8, 64), 1)
        e8 = (ri == (ci >> 3)).astype(jnp.float32)
        h1 = (jnp.dot(gn0, e8 * w1_ref[0:1, :],
                      preferred_element_type=jnp.float32)
              + jnp.dot(gn1, e8 * w1_ref[1:2, :],
                        preferred_element_type=jnp.float32)
              + b1_ref[...])
        h1 = jnp.where(h1 > 0, h1, jnp.exp(jnp.minimum(h1, 0.0)) - 1.0)
        h2p = jnp.dot(h1, w2_ref[...], preferred_element_type=jnp.float32)
        a2s = jnp.sum(h2p * as2_ref[...], axis=1, keepdims=True)
        a2d = jnp.sum(h2p * ad2_ref[...], axis=1, keepdims=True)
        t2_ref[...] = jnp.concatenate(
            [a2s, a2d, h2p, jnp.zeros_like(h2p), jnp.zeros_like(h2p)],
            axis=1)

    return pl.pallas_call(
        body,
        grid=(np_ // blk,),
        in_specs=[
            pl.BlockSpec((2, blk, 16), lambda i: (0, i, 0)),
            pl.BlockSpec((blk, 8), lambda i: (i, 0)),
            pl.BlockSpec((2, 64), lambda i: (0, 0)),
            pl.BlockSpec((1, 64), lambda i: (0, 0)),
            pl.BlockSpec((64, 2), lambda i: (0, 0)),
            pl.BlockSpec((1, 2), lambda i: (0, 0)),
            pl.BlockSpec((1, 2), lambda i: (0, 0)),
        ],
        out_specs=pl.BlockSpec((blk, 8), lambda i: (i, 0)),
        out_shape=jax.ShapeDtypeStruct((np_, 8), jnp.float32),
    )(g_p, rden, w1, b1r, w2, as2, ad2)


def _tc4(acc2_p, b2r, n, np_, blk):
    def body(ap_ref, b2_ref, h2_ref, rdt_ref):
        i = pl.program_id(0)
        a = ap_ref[0] + ap_ref[1]
        rows = i * blk + lax.broadcasted_iota(jnp.int32, (blk, 1), 0)
        rd2 = jnp.where(rows < n, 1.0 / (a[:, 0:1] + 1e-16), 0.0)
        h2_ref[...] = a[:, 1:3] * rd2 + b2_ref[...]
        rdt_ref[...] = jnp.broadcast_to(rd2, (blk, 8))

    return pl.pallas_call(
        body,
        grid=(np_ // blk,),
        in_specs=[
            pl.BlockSpec((2, blk, 8), lambda i: (0, i, 0)),
            pl.BlockSpec((1, 2), lambda i: (0, 0)),
        ],
        out_specs=[
            pl.BlockSpec((blk, 2), lambda i: (i, 0)),
            pl.BlockSpec((blk, 8), lambda i: (i, 0)),
        ],
        out_shape=[
            jax.ShapeDtypeStruct((np_, 2), jnp.float32),
            jax.ShapeDtypeStruct((np_, 8), jnp.float32),
        ],
    )(acc2_p, b2r)


def _tcslice8(a, rows):
    ep = a.shape[0]
    blk = 4096
    while -(-rows // blk) * blk > ep:
        blk //= 2

    def body(a_ref, o_ref):
        o_ref[...] = a_ref[...]

    return pl.pallas_call(
        body,
        grid=(_cdiv(rows, blk),),
        in_specs=[pl.BlockSpec((blk, 8), lambda i: (i, 0))],
        out_specs=pl.BlockSpec((blk, 8), lambda i: (i, 0)),
        out_shape=jax.ShapeDtypeStruct((rows, 8), jnp.float32),
    )(a)


def _sca(src, dst, t1s, t1d, z8, np_, cpw):
    ep = src.shape[0]
    npt = np_ // NS
    zr = z8.shape[0]
    nz = npt // zr

    @functools.partial(
        pl.kernel,
        out_type=(jax.ShapeDtypeStruct((ep, 8), jnp.float32),
                  jax.ShapeDtypeStruct((NC, np_, 8), jnp.float32)),
        mesh=_mesh(),
        compiler_params=pltpu.CompilerParams(needs_layout_passes=False, use_tc_tiling_on_sc=False),
        scratch_types=[
            pltpu.VMEM((CH,), jnp.int32), pltpu.VMEM((CH,), jnp.int32),
            pltpu.VMEM((CH, 8), jnp.float32), pltpu.VMEM((CH, 8), jnp.float32),
            pltpu.VMEM((CH, 8), jnp.float32), pltpu.VMEM((zr, 8), jnp.float32),
            pltpu.VMEM_SHARED((np_, 8), jnp.float32),
            pltpu.SemaphoreType.DMA, pltpu.SemaphoreType.DMA,
        ],
    )
    def k(src_h, dst_h, t1s_h, t1d_h, z8_h, s_out_h, den_out_h,
          src_v, dst_v, gs_v, gd_v, s_v, zb_v, den_sh, sem1, sem2):
        c = lax.axis_index("c")
        t = lax.axis_index("s")
        wid = t * NC + c
        lane = lax.iota(jnp.int32, 16)

        pltpu.sync_copy(z8_h, zb_v)

        def zloop(r, u):
            pltpu.sync_copy(zb_v, den_sh.at[pl.ds(t * npt + r * zr, zr), :])
            return u
        lax.fori_loop(0, nz, zloop, 0)
        plsc.subcore_barrier()

        def chunk(kk, u):
            base = (wid * cpw + kk) * CH
            pltpu.sync_copy(src_h.at[pl.ds(base, CH)], src_v)
            pltpu.sync_copy(dst_h.at[pl.ds(base, CH)], dst_v)
            cp1 = pltpu.make_async_copy(t1s_h.at[src_v], gs_v, sem1)
            cp1.start()
            cp2 = pltpu.make_async_copy(t1d_h.at[dst_v], gd_v, sem2)
            cp2.start()
            cp1.wait()
            cp2.wait()

            def inner(i, v):
                e16 = 2 * i + (lane >> 3)
                h16 = lane & 7
                a = (plsc.load_gather(gs_v, [e16, h16])
                     + plsc.load_gather(gd_v, [e16, h16]))
                sv = jnp.exp(jnp.maximum(a, 0.2 * a))
                plsc.store_scatter(s_v, [e16, h16], sv)
                return v
            lax.fori_loop(0, CH * 8 // 16, inner, 0, unroll=4)

            pltpu.sync_copy(s_v, s_out_h.at[pl.ds(base, CH), :])
            pltpu.sync_copy(s_v, den_sh.at[dst_v], add=True)
            return u
        lax.fori_loop(0, cpw, chunk, 0)
        plsc.subcore_barrier()

        def rloop(r, u):
            sl = pl.ds(t * npt + r * zr, zr)
            pltpu.sync_copy(den_sh.at[sl, :], zb_v)
            pltpu.sync_copy(zb_v, den_out_h.at[c, sl, :])
            return u
        lax.fori_loop(0, nz, rloop, 0)

    return k(src, dst, t1s, t1d, z8)


def _scb(src, dst, s_in, rden, x_p, z16, np_, cpw):
    ep = src.shape[0]
    npt = np_ // NS
    zr = z16.shape[0]
    nz = npt // zr

    @functools.partial(
        pl.kernel,
        out_type=(jax.ShapeDtypeStruct((ep, 8), jnp.float32),
                  jax.ShapeDtypeStruct((NC, np_, 16), jnp.float32)),
        mesh=_mesh(),
        compiler_params=pltpu.CompilerParams(needs_layout_passes=False, use_tc_tiling_on_sc=False),
        scratch_types=[
            pltpu.VMEM((CHB,), jnp.int32), pltpu.VMEM((CHB,), jnp.int32),
            pltpu.VMEM((CHB, 8), jnp.float32), pltpu.VMEM((CHB, 8), jnp.float32),
            pltpu.VMEM((CHB, 8), jnp.float32), pltpu.VMEM((CHB, 8), jnp.float32),
            pltpu.VMEM((CHB, 16), jnp.float32),
            pltpu.VMEM((zr, 16), jnp.float32),
            pltpu.VMEM_SHARED((np_, 16), jnp.float32),
            pltpu.SemaphoreType.DMA, pltpu.SemaphoreType.DMA,
        ],
    )
    def k(src_h, dst_h, s_in_h, rden_h, x_h, z16_h, attn_h, g_out_h,
          src_v, dst_v, s_v, rd_v, at_v, x_v, p_v, zb_v,
          g_sh, sem1, sem2):
        c = lax.axis_index("c")
        t = lax.axis_index("s")
        wid = t * NC + c
        lane = lax.iota(jnp.int32, 16)

        pltpu.sync_copy(z16_h, zb_v)

        def zloop(r, u):
            pltpu.sync_copy(zb_v, g_sh.at[pl.ds(t * npt + r * zr, zr), :])
            return u
        lax.fori_loop(0, nz, zloop, 0)
        plsc.subcore_barrier()

        def chunk(kk, u):
            base = (wid * cpw + kk) * CHB
            pltpu.sync_copy(src_h.at[pl.ds(base, CHB)], src_v)
            pltpu.sync_copy(dst_h.at[pl.ds(base, CHB)], dst_v)
            pltpu.sync_copy(s_in_h.at[pl.ds(base, CHB), :], s_v)
            cp1 = pltpu.make_async_copy(rden_h.at[dst_v], rd_v, sem1)
            cp1.start()
            cp2 = pltpu.make_async_copy(x_h.at[src_v], x_v, sem2)
            cp2.start()
            cp1.wait()
            cp2.wait()

            def inner1(i, v):
                e16 = 2 * i + (lane >> 3)
                h16 = lane & 7
                at = (plsc.load_gather(s_v, [e16, h16])
                      * plsc.load_gather(rd_v, [e16, h16]))
                plsc.store_scatter(at_v, [e16, h16], at)
                return v
            lax.fori_loop(0, CHB * 8 // 16, inner1, 0, unroll=4)

            def inner2(e, v):
                eb = lane * 0 + e
                p = (plsc.load_gather(s_v, [eb, lane & 7])
                     * plsc.load_gather(x_v, [eb, lane >> 3]))
                plsc.store_scatter(p_v, [eb, lane], p)
                return v
            lax.fori_loop(0, CHB, inner2, 0, unroll=4)

            pltpu.sync_copy(at_v, attn_h.at[pl.ds(base, CHB), :])
            pltpu.sync_copy(p_v, g_sh.at[dst_v], add=True)
            return u
        lax.fori_loop(0, cpw, chunk, 0)
        plsc.subcore_barrier()

        def rloop(r, u):
            sl = pl.ds(t * npt + r * zr, zr)
            pltpu.sync_copy(g_sh.at[sl, :], zb_v)
            pltpu.sync_copy(zb_v, g_out_h.at[c, sl, :])
            return u
        lax.fori_loop(0, nz, rloop, 0)

    return k(src, dst, s_in, rden, x_p, z16)


def _l2a(src, dst, t2, z8, np_, cpw):
    ep = src.shape[0]
    npt = np_ // NS
    zr = z8.shape[0]
    nz = npt // zr

    @functools.partial(
        pl.kernel,
        out_type=(jax.ShapeDtypeStruct((ep,), jnp.float32),
                  jax.ShapeDtypeStruct((NC, np_, 8), jnp.float32)),
        mesh=_mesh(),
        compiler_params=pltpu.CompilerParams(needs_layout_passes=False, use_tc_tiling_on_sc=False),
        scratch_types=[
            pltpu.VMEM((CHB,), jnp.int32), pltpu.VMEM((CHB,), jnp.int32),
            pltpu.VMEM((CHB, 8), jnp.float32), pltpu.VMEM((CHB, 8), jnp.float32),
            pltpu.VMEM((CHB,), jnp.float32), pltpu.VMEM((CHB, 8), jnp.float32),
            pltpu.VMEM((zr, 8), jnp.float32),
            pltpu.VMEM_SHARED((np_, 8), jnp.float32),
            pltpu.VMEM_SHARED((np_, 8), jnp.float32),
            pltpu.SemaphoreType.DMA, pltpu.SemaphoreType.DMA,
        ],
    )
    def k(src_h, dst_h, t2_h, z8_h, s2_out_h, acc_out_h,
          src_v, dst_v, g2s_v, g2d_v, s2_v, p2_v, zb_v,
          t2_sh, acc_sh, sem1, sem2):
        c = lax.axis_index("c")
        t = lax.axis_index("s")
        wid = t * NC + c
        lane = lax.iota(jnp.int32, 16)

        pltpu.sync_copy(z8_h, zb_v)

        def zloop(r, u):
            pltpu.sync_copy(zb_v, acc_sh.at[pl.ds(t * npt + r * zr, zr), :])
            return u
        lax.fori_loop(0, nz, zloop, 0)

        def sloop(r, u):
            sl = pl.ds(t * npt + r * zr, zr)
            pltpu.sync_copy(t2_h.at[sl, :], zb_v)
            pltpu.sync_copy(zb_v, t2_sh.at[sl, :])
            return u
        lax.fori_loop(0, nz, sloop, 0)
        plsc.subcore_barrier()

        def chunk(kk, u):
            base = (wid * cpw + kk) * CHB
            pltpu.sync_copy(src_h.at[pl.ds(base, CHB)], src_v)
            pltpu.sync_copy(dst_h.at[pl.ds(base, CHB)], dst_v)
            cp1 = pltpu.make_async_copy(t2_sh.at[src_v], g2s_v, sem1)
            cp1.start()
            cp2 = pltpu.make_async_copy(t2_sh.at[dst_v], g2d_v, sem2)
            cp2.start()
            cp1.wait()
            cp2.wait()

            def inner1(i, v):
                e16 = 16 * i + lane
                a = (plsc.load_gather(g2s_v, [e16, lane * 0])
                     + plsc.load_gather(g2d_v, [e16, lane * 0 + 1]))
                s2 = jnp.exp(jnp.maximum(a, 0.2 * a))
                s2_v[pl.ds(16 * i, 16)] = s2
                return v
            lax.fori_loop(0, CHB // 16, inner1, 0, unroll=4)

            def inner2(i, v):
                er = 2 * i + (lane >> 3)
                col = lane & 7
                s2g = plsc.load_gather(s2_v, [er])
                hv = plsc.load_gather(g2s_v, [er, jnp.minimum(col + 1, 7)])
                p2 = jnp.where(col == 0, s2g, s2g * hv)
                plsc.store_scatter(p2_v, [er, col], p2)
                return v
            lax.fori_loop(0, CHB // 2, inner2, 0, unroll=4)

            pltpu.sync_copy(s2_v, s2_out_h.at[pl.ds(base, CHB)])
            pltpu.sync_copy(p2_v, acc_sh.at[dst_v], add=True)
            return u
        lax.fori_loop(0, cpw, chunk, 0)
        plsc.subcore_barrier()

        def rloop(r, u):
            sl = pl.ds(t * npt + r * zr, zr)
            pltpu.sync_copy(acc_sh.at[sl, :], zb_v)
            pltpu.sync_copy(zb_v, acc_out_h.at[c, sl, :])
            return u
        lax.fori_loop(0, nz, rloop, 0)

    return k(src, dst, t2, z8)


def _l2b(dst, s2, rdt, np_, cpw):
    ep = dst.shape[0]
    npt = np_ // NS
    zr = npt // 4
    nz = 4

    @functools.partial(
        pl.kernel,
        out_type=jax.ShapeDtypeStruct((ep,), jnp.float32),
        mesh=_mesh(),
        compiler_params=pltpu.CompilerParams(needs_layout_passes=False, use_tc_tiling_on_sc=False),
        scratch_types=[
            pltpu.VMEM((CH,), jnp.int32), pltpu.VMEM((CH,), jnp.float32),
            pltpu.VMEM((CH, 8), jnp.float32), pltpu.VMEM((CH,), jnp.float32),
            pltpu.VMEM((zr, 8), jnp.float32),
            pltpu.VMEM_SHARED((np_, 8), jnp.float32),
            pltpu.SemaphoreType.DMA,
        ],
    )
    def k(dst_h, s2_h, rdt_h, attn_h,
          dst_v, s2_v, rg_v, at_v, zb_v, rd_sh, sem1):
        c = lax.axis_index("c")
        t = lax.axis_index("s")
        wid = t * NC + c
        lane = lax.iota(jnp.int32, 16)

        def sloop(r, u):
            sl = pl.ds(t * npt + r * zr, zr)
            pltpu.sync_copy(rdt_h.at[sl, :], zb_v)
            pltpu.sync_copy(zb_v, rd_sh.at[sl, :])
            return u
        lax.fori_loop(0, nz, sloop, 0)
        plsc.subcore_barrier()

        def chunk(kk, u):
            base = (wid * cpw + kk) * CH
            pltpu.sync_copy(dst_h.at[pl.ds(base, CH)], dst_v)
            pltpu.sync_copy(s2_h.at[pl.ds(base, CH)], s2_v)
            pltpu.make_async_copy(rd_sh.at[dst_v], rg_v, sem1).start()
            pltpu.make_async_copy(rd_sh.at[dst_v], rg_v, sem1).wait()

            def inner(i, v):
                e16 = 16 * i + lane
                rv = plsc.load_gather(rg_v, [e16, lane * 0])
                at_v[pl.ds(16 * i, 16)] = s2_v[pl.ds(16 * i, 16)] * rv
                return v
            lax.fori_loop(0, CH // 16, inner, 0, unroll=4)

            pltpu.sync_copy(at_v, attn_h.at[pl.ds(base, CH)])
            return u
        lax.fori_loop(0, cpw, chunk, 0)

    return k(dst, s2, rdt)


def kernel(x, edge_index, W1, att_src1, att_dst1, b1,
           W2, att_src2, att_dst2, b2):
    n = x.shape[0]
    e = edge_index.shape[1]
    etot = e + n

    cpw = _cdiv(etot, NW * CH)
    ep = NW * cpw * CH
    np_ = BLK * _cdiv(n + 1, BLK)
    npt = np_ // NS
    zr = npt // 4

    loop = jnp.arange(n, dtype=jnp.int32)
    src = jnp.concatenate([edge_index[0].astype(jnp.int32), loop,
                           jnp.zeros((ep - etot,), jnp.int32)])
    dst = jnp.concatenate([edge_index[1].astype(jnp.int32), loop,
                           jnp.full((ep - etot,), n, jnp.int32)])
    x_p = jnp.zeros((np_, 2), jnp.float32).at[:n].set(x)
    x8_p = jnp.zeros((np_, 8), jnp.float32).at[:n, 0:2].set(x)
    z8 = jnp.zeros((zr, 8), jnp.float32)
    z16 = jnp.zeros((npt // 16, 16), jnp.float32)
    as1 = att_src1.reshape(64, 1)
    ad1 = att_dst1.reshape(64, 1)
    b1r = b1.reshape(1, 64)
    b2r = b2.reshape(1, 2)
    as2 = att_src2.reshape(1, 2)
    ad2 = att_dst2.reshape(1, 2)

    t1s, t1d = _tc1(x_p, W1, as1, ad1, np_, BLK)
    s1, den_p = _sca(src, dst, t1s, t1d, z8, np_, cpw)
    rden = _tc2(den_p, n, np_, BLK)
    attn1, g_p = _scb(src, dst, s1, rden, x8_p, z16, np_,
                      cpw * (CH // CHB))
    t2 = _tc3(g_p, rden, W1, b1r, W2, as2, ad2, np_, BLK)
    s2, acc2_p = _l2a(src, dst, t2, z8, np_, cpw * (CH // CHB))
    h2_p, rdt = _tc4(acc2_p, b2r, n, np_, BLK)
    attn2 = _l2b(dst, s2, rdt, np_, cpw)

    return (h2_p[:n],
            (_tcslice8(attn1, etot), attn2[:etot].reshape(etot, 1)))


# speedup vs baseline: 1.2137x; 1.2137x over previous
import functools

import jax
import jax.numpy as jnp
from jax import lax
from jax.experimental import pallas as pl
from jax.experimental.pallas import tpu as pltpu
from jax.experimental.pallas import tpu_sc as plsc

NC, NS, LANES = 2, 16, 16
NW = NC * NS
CH = 1024
CHB = 256
BLK = 2048


def _cdiv(a, b):
    return (a + b - 1) // b


def _mesh():
    return plsc.VectorSubcoreMesh(
        core_axis_name="c", subcore_axis_name="s",
        num_cores=NC, num_subcores=NS)


def _tc1(x_p, w1, as1, ad1, np_, blk):
    def body(x_ref, w1_ref, as_ref, ad_ref, ts_ref, td_ref):
        h = jnp.dot(x_ref[...], w1_ref[...],
                    preferred_element_type=jnp.float32)
        ri = lax.broadcasted_iota(jnp.int32, (64, 8), 0)
        ci = lax.broadcasted_iota(jnp.int32, (64, 8), 1)
        msk = (ci == (ri >> 3)).astype(jnp.float32)
        ts_ref[...] = jnp.dot(h, as_ref[...] * msk,
                              preferred_element_type=jnp.float32)
        td_ref[...] = jnp.dot(h, ad_ref[...] * msk,
                              preferred_element_type=jnp.float32)

    grid = (np_ // blk,)
    return pl.pallas_call(
        body,
        grid=grid,
        in_specs=[
            pl.BlockSpec((blk, 2), lambda i: (i, 0)),
            pl.BlockSpec((2, 64), lambda i: (0, 0)),
            pl.BlockSpec((64, 1), lambda i: (0, 0)),
            pl.BlockSpec((64, 1), lambda i: (0, 0)),
        ],
        out_specs=[
            pl.BlockSpec((blk, 8), lambda i: (i, 0)),
            pl.BlockSpec((blk, 8), lambda i: (i, 0)),
        ],
        out_shape=[
            jax.ShapeDtypeStruct((np_, 8), jnp.float32),
            jax.ShapeDtypeStruct((np_, 8), jnp.float32),
        ],
    )(x_p, w1, as1, ad1)


def _tc2(den_p, n, np_, blk):
    def body(dp_ref, rd_ref):
        i = pl.program_id(0)
        d = dp_ref[0] + dp_ref[1]
        rows = i * blk + lax.broadcasted_iota(jnp.int32, (blk, 8), 0)
        rd_ref[...] = jnp.where(rows < n, 1.0 / (d + 1e-16), 0.0)

    return pl.pallas_call(
        body,
        grid=(np_ // blk,),
        in_specs=[pl.BlockSpec((2, blk, 8), lambda i: (0, i, 0))],
        out_specs=pl.BlockSpec((blk, 8), lambda i: (i, 0)),
        out_shape=jax.ShapeDtypeStruct((np_, 8), jnp.float32),
    )(den_p)


def _tc3(g_p, rden, w1, b1r, w2, as2, ad2, np_, blk):
    def body(gp_ref, rd_ref, w1_ref, b1_ref, w2_ref, as2_ref, ad2_ref,
             t2_ref):
        g = gp_ref[0] + gp_ref[1]
        rd = rd_ref[...]
        gn0 = g[:, 0:8] * rd
        gn1 = g[:, 8:16] * rd
        ri = lax.broadcasted_iota(jnp.int32, (8, 64), 0)
        ci = lax.broadcasted_iota(jnp.int32, (8, 64), 1)
        e8 = (ri == (ci >> 3)).astype(jnp.float32)
        h1 = (jnp.dot(gn0, e8 * w1_ref[0:1, :],
                      preferred_element_type=jnp.float32)
              + jnp.dot(gn1, e8 * w1_ref[1:2, :],
                        preferred_element_type=jnp.float32)
              + b1_ref[...])
        h1 = jnp.where(h1 > 0, h1, jnp.exp(jnp.minimum(h1, 0.0)) - 1.0)
        h2p = jnp.dot(h1, w2_ref[...], preferred_element_type=jnp.float32)
        a2s = jnp.sum(h2p * as2_ref[...], axis=1, keepdims=True)
        a2d = jnp.sum(h2p * ad2_ref[...], axis=1, keepdims=True)
        t2_ref[...] = jnp.concatenate(
            [a2s, a2d, h2p, jnp.zeros_like(h2p), jnp.zeros_like(h2p)],
            axis=1)

    return pl.pallas_call(
        body,
        grid=(np_ // blk,),
        in_specs=[
            pl.BlockSpec((2, blk, 16), lambda i: (0, i, 0)),
            pl.BlockSpec((blk, 8), lambda i: (i, 0)),
            pl.BlockSpec((2, 64), lambda i: (0, 0)),
            pl.BlockSpec((1, 64), lambda i: (0, 0)),
            pl.BlockSpec((64, 2), lambda i: (0, 0)),
            pl.BlockSpec((1, 2), lambda i: (0, 0)),
            pl.BlockSpec((1, 2), lambda i: (0, 0)),
        ],
        out_specs=pl.BlockSpec((blk, 8), lambda i: (i, 0)),
        out_shape=jax.ShapeDtypeStruct((np_, 8), jnp.float32),
    )(g_p, rden, w1, b1r, w2, as2, ad2)


def _tc4(acc2_p, b2r, n, np_, blk):
    def body(ap_ref, b2_ref, h2_ref, rdt_ref):
        i = pl.program_id(0)
        a = ap_ref[0] + ap_ref[1]
        rows = i * blk + lax.broadcasted_iota(jnp.int32, (blk, 1), 0)
        rd2 = jnp.where(rows < n, 1.0 / (a[:, 0:1] + 1e-16), 0.0)
        h2_ref[...] = a[:, 1:3] * rd2 + b2_ref[...]
        rdt_ref[...] = jnp.broadcast_to(rd2, (blk, 8))

    return pl.pallas_call(
        body,
        grid=(np_ // blk,),
        in_specs=[
            pl.BlockSpec((2, blk, 8), lambda i: (0, i, 0)),
            pl.BlockSpec((1, 2), lambda i: (0, 0)),
        ],
        out_specs=[
            pl.BlockSpec((blk, 2), lambda i: (i, 0)),
            pl.BlockSpec((blk, 8), lambda i: (i, 0)),
        ],
        out_shape=[
            jax.ShapeDtypeStruct((np_, 2), jnp.float32),
            jax.ShapeDtypeStruct((np_, 8), jnp.float32),
        ],
    )(acc2_p, b2r)


def _sca(src, dst, t1s, t1d, z8, np_, cpw):
    ep = src.shape[0]
    npt = np_ // NS
    zr = z8.shape[0]
    nz = npt // zr

    @functools.partial(
        pl.kernel,
        out_type=(jax.ShapeDtypeStruct((ep, 8), jnp.float32),
                  jax.ShapeDtypeStruct((NC, np_, 8), jnp.float32)),
        mesh=_mesh(),
        compiler_params=pltpu.CompilerParams(needs_layout_passes=False, use_tc_tiling_on_sc=False),
        scratch_types=[
            pltpu.VMEM((CH,), jnp.int32), pltpu.VMEM((CH,), jnp.int32),
            pltpu.VMEM((CH, 8), jnp.float32), pltpu.VMEM((CH, 8), jnp.float32),
            pltpu.VMEM((CH, 8), jnp.float32), pltpu.VMEM((zr, 8), jnp.float32),
            pltpu.VMEM_SHARED((np_, 8), jnp.float32),
            pltpu.SemaphoreType.DMA, pltpu.SemaphoreType.DMA,
        ],
    )
    def k(src_h, dst_h, t1s_h, t1d_h, z8_h, s_out_h, den_out_h,
          src_v, dst_v, gs_v, gd_v, s_v, zb_v, den_sh, sem1, sem2):
        c = lax.axis_index("c")
        t = lax.axis_index("s")
        wid = t * NC + c
        lane = lax.iota(jnp.int32, 16)

        pltpu.sync_copy(z8_h, zb_v)

        def zloop(r, u):
            pltpu.sync_copy(zb_v, den_sh.at[pl.ds(t * npt + r * zr, zr), :])
            return u
        lax.fori_loop(0, nz, zloop, 0)
        plsc.subcore_barrier()

        def chunk(kk, u):
            base = (wid * cpw + kk) * CH
            pltpu.sync_copy(src_h.at[pl.ds(base, CH)], src_v)
            pltpu.sync_copy(dst_h.at[pl.ds(base, CH)], dst_v)
            cp1 = pltpu.make_async_copy(t1s_h.at[src_v], gs_v, sem1)
            cp1.start()
            cp2 = pltpu.make_async_copy(t1d_h.at[dst_v], gd_v, sem2)
            cp2.start()
            cp1.wait()
            cp2.wait()

            def inner(i, v):
                e16 = 2 * i + (lane >> 3)
                h16 = lane & 7
                a = (plsc.load_gather(gs_v, [e16, h16])
                     + plsc.load_gather(gd_v, [e16, h16]))
                sv = jnp.exp(jnp.maximum(a, 0.2 * a))
                plsc.store_scatter(s_v, [e16, h16], sv)
                return v
            lax.fori_loop(0, CH * 8 // 16, inner, 0, unroll=4)

            pltpu.sync_copy(s_v, s_out_h.at[pl.ds(base, CH), :])
            pltpu.sync_copy(s_v, den_sh.at[dst_v], add=True)
            return u
        lax.fori_loop(0, cpw, chunk, 0)
        plsc.subcore_barrier()

        def rloop(r, u):
            sl = pl.ds(t * npt + r * zr, zr)
            pltpu.sync_copy(den_sh.at[sl, :], zb_v)
            pltpu.sync_copy(zb_v, den_out_h.at[c, sl, :])
            return u
        lax.fori_loop(0, nz, rloop, 0)

    return k(src, dst, t1s, t1d, z8)


def _scb(src, dst, s_in, rden, x_p, z16, np_, cpw, etot):
    ep = src.shape[0]
    npt = np_ // NS
    zr = z16.shape[0]
    nz = npt // zr
    tail = etot % CHB

    @functools.partial(
        pl.kernel,
        out_type=(jax.ShapeDtypeStruct((etot, 8), jnp.float32),
                  jax.ShapeDtypeStruct((NC, np_, 16), jnp.float32)),
        mesh=_mesh(),
        compiler_params=pltpu.CompilerParams(needs_layout_passes=False, use_tc_tiling_on_sc=False),
        scratch_types=[
            pltpu.VMEM((CHB,), jnp.int32), pltpu.VMEM((CHB,), jnp.int32),
            pltpu.VMEM((CHB, 8), jnp.float32), pltpu.VMEM((CHB, 8), jnp.float32),
            pltpu.VMEM((CHB, 8), jnp.float32), pltpu.VMEM((CHB, 8), jnp.float32),
            pltpu.VMEM((CHB, 16), jnp.float32),
            pltpu.VMEM((zr, 16), jnp.float32),
            pltpu.VMEM_SHARED((np_, 16), jnp.float32),
            pltpu.SemaphoreType.DMA, pltpu.SemaphoreType.DMA,
        ],
    )
    def k(src_h, dst_h, s_in_h, rden_h, x_h, z16_h, attn_h, g_out_h,
          src_v, dst_v, s_v, rd_v, at_v, x_v, p_v, zb_v,
          g_sh, sem1, sem2):
        c = lax.axis_index("c")
        t = lax.axis_index("s")
        wid = t * NC + c
        lane = lax.iota(jnp.int32, 16)

        pltpu.sync_copy(z16_h, zb_v)

        def zloop(r, u):
            pltpu.sync_copy(zb_v, g_sh.at[pl.ds(t * npt + r * zr, zr), :])
            return u
        lax.fori_loop(0, nz, zloop, 0)
        plsc.subcore_barrier()

        def chunk(kk, u):
            base = (wid * cpw + kk) * CHB
            pltpu.sync_copy(src_h.at[pl.ds(base, CHB)], src_v)
            pltpu.sync_copy(dst_h.at[pl.ds(base, CHB)], dst_v)
            pltpu.sync_copy(s_in_h.at[pl.ds(base, CHB), :], s_v)
            cp1 = pltpu.make_async_copy(rden_h.at[dst_v], rd_v, sem1)
            cp1.start()
            cp2 = pltpu.make_async_copy(x_h.at[src_v], x_v, sem2)
            cp2.start()
            cp1.wait()
            cp2.wait()

            def inner1(i, v):
                e16 = 2 * i + (lane >> 3)
                h16 = lane & 7
                at = (plsc.load_gather(s_v, [e16, h16])
                      * plsc.load_gather(rd_v, [e16, h16]))
                plsc.store_scatter(at_v, [e16, h16], at)
                return v
            lax.fori_loop(0, CHB * 8 // 16, inner1, 0, unroll=4)

            def inner2(e, v):
                eb = lane * 0 + e
                p = (plsc.load_gather(s_v, [eb, lane & 7])
                     * plsc.load_gather(x_v, [eb, lane >> 3]))
                plsc.store_scatter(p_v, [eb, lane], p)
                return v
            lax.fori_loop(0, CHB, inner2, 0, unroll=4)

            @pl.when(base + CHB <= etot)
            def _():
                pltpu.sync_copy(at_v, attn_h.at[pl.ds(base, CHB), :])

            if tail:
                @pl.when(base + CHB - etot == CHB - tail)
                def _():
                    pltpu.sync_copy(at_v.at[pl.ds(0, tail), :],
                                    attn_h.at[pl.ds(etot - tail, tail), :])
            pltpu.sync_copy(p_v, g_sh.at[dst_v], add=True)
            return u
        lax.fori_loop(0, cpw, chunk, 0)
        plsc.subcore_barrier()

        def rloop(r, u):
            sl = pl.ds(t * npt + r * zr, zr)
            pltpu.sync_copy(g_sh.at[sl, :], zb_v)
            pltpu.sync_copy(zb_v, g_out_h.at[c, sl, :])
            return u
        lax.fori_loop(0, nz, rloop, 0)

    return k(src, dst, s_in, rden, x_p, z16)


def _l2a(src, dst, t2, z8, np_, cpw):
    ep = src.shape[0]
    npt = np_ // NS
    zr = z8.shape[0]
    nz = npt // zr

    @functools.partial(
        pl.kernel,
        out_type=(jax.ShapeDtypeStruct((ep,), jnp.float32),
                  jax.ShapeDtypeStruct((NC, np_, 8), jnp.float32)),
        mesh=_mesh(),
        compiler_params=pltpu.CompilerParams(needs_layout_passes=False, use_tc_tiling_on_sc=False),
        scratch_types=[
            pltpu.VMEM((CHB,), jnp.int32), pltpu.VMEM((CHB,), jnp.int32),
            pltpu.VMEM((CHB, 8), jnp.float32), pltpu.VMEM((CHB, 8), jnp.float32),
            pltpu.VMEM((CHB,), jnp.float32), pltpu.VMEM((CHB, 8), jnp.float32),
            pltpu.VMEM((zr, 8), jnp.float32),
            pltpu.VMEM_SHARED((np_, 8), jnp.float32),
            pltpu.VMEM_SHARED((np_, 8), jnp.float32),
            pltpu.SemaphoreType.DMA, pltpu.SemaphoreType.DMA,
        ],
    )
    def k(src_h, dst_h, t2_h, z8_h, s2_out_h, acc_out_h,
          src_v, dst_v, g2s_v, g2d_v, s2_v, p2_v, zb_v,
          t2_sh, acc_sh, sem1, sem2):
        c = lax.axis_index("c")
        t = lax.axis_index("s")
        wid = t * NC + c
        lane = lax.iota(jnp.int32, 16)

        pltpu.sync_copy(z8_h, zb_v)

        def zloop(r, u):
            pltpu.sync_copy(zb_v, acc_sh.at[pl.ds(t * npt + r * zr, zr), :])
            return u
        lax.fori_loop(0, nz, zloop, 0)

        def sloop(r, u):
            sl = pl.ds(t * npt + r * zr, zr)
            pltpu.sync_copy(t2_h.at[sl, :], zb_v)
            pltpu.sync_copy(zb_v, t2_sh.at[sl, :])
            return u
        lax.fori_loop(0, nz, sloop, 0)
        plsc.subcore_barrier()

        def chunk(kk, u):
            base = (wid * cpw + kk) * CHB
            pltpu.sync_copy(src_h.at[pl.ds(base, CHB)], src_v)
            pltpu.sync_copy(dst_h.at[pl.ds(base, CHB)], dst_v)
            cp1 = pltpu.make_async_copy(t2_sh.at[src_v], g2s_v, sem1)
            cp1.start()
            cp2 = pltpu.make_async_copy(t2_sh.at[dst_v], g2d_v, sem2)
            cp2.start()
            cp1.wait()
            cp2.wait()

            def inner1(i, v):
                e16 = 16 * i + lane
                a = (plsc.load_gather(g2s_v, [e16, lane * 0])
                     + plsc.load_gather(g2d_v, [e16, lane * 0 + 1]))
                s2 = jnp.exp(jnp.maximum(a, 0.2 * a))
                s2_v[pl.ds(16 * i, 16)] = s2
                return v
            lax.fori_loop(0, CHB // 16, inner1, 0, unroll=4)

            def inner2(i, v):
                er = 2 * i + (lane >> 3)
                col = lane & 7
                s2g = plsc.load_gather(s2_v, [er])
                hv = plsc.load_gather(g2s_v, [er, jnp.minimum(col + 1, 7)])
                p2 = jnp.where(col == 0, s2g, s2g * hv)
                plsc.store_scatter(p2_v, [er, col], p2)
                return v
            lax.fori_loop(0, CHB // 2, inner2, 0, unroll=4)

            pltpu.sync_copy(s2_v, s2_out_h.at[pl.ds(base, CHB)])
            pltpu.sync_copy(p2_v, acc_sh.at[dst_v], add=True)
            return u
        lax.fori_loop(0, cpw, chunk, 0)
        plsc.subcore_barrier()

        def rloop(r, u):
            sl = pl.ds(t * npt + r * zr, zr)
            pltpu.sync_copy(acc_sh.at[sl, :], zb_v)
            pltpu.sync_copy(zb_v, acc_out_h.at[c, sl, :])
            return u
        lax.fori_loop(0, nz, rloop, 0)

    return k(src, dst, t2, z8)


def _l2b(dst, s2, rdt, np_, cpw, etot):
    ep = dst.shape[0]
    npt = np_ // NS
    zr = npt // 4
    nz = 4
    tail = etot % CH

    @functools.partial(
        pl.kernel,
        out_type=jax.ShapeDtypeStruct((etot,), jnp.float32),
        mesh=_mesh(),
        compiler_params=pltpu.CompilerParams(needs_layout_passes=False, use_tc_tiling_on_sc=False),
        scratch_types=[
            pltpu.VMEM((CH,), jnp.int32), pltpu.VMEM((CH,), jnp.float32),
            pltpu.VMEM((CH, 8), jnp.float32), pltpu.VMEM((CH,), jnp.float32),
            pltpu.VMEM((zr, 8), jnp.float32),
            pltpu.VMEM_SHARED((np_, 8), jnp.float32),
            pltpu.SemaphoreType.DMA,
        ],
    )
    def k(dst_h, s2_h, rdt_h, attn_h,
          dst_v, s2_v, rg_v, at_v, zb_v, rd_sh, sem1):
        c = lax.axis_index("c")
        t = lax.axis_index("s")
        wid = t * NC + c
        lane = lax.iota(jnp.int32, 16)

        def sloop(r, u):
            sl = pl.ds(t * npt + r * zr, zr)
            pltpu.sync_copy(rdt_h.at[sl, :], zb_v)
            pltpu.sync_copy(zb_v, rd_sh.at[sl, :])
            return u
        lax.fori_loop(0, nz, sloop, 0)
        plsc.subcore_barrier()

        def chunk(kk, u):
            base = (wid * cpw + kk) * CH
            pltpu.sync_copy(dst_h.at[pl.ds(base, CH)], dst_v)
            pltpu.sync_copy(s2_h.at[pl.ds(base, CH)], s2_v)
            pltpu.make_async_copy(rd_sh.at[dst_v], rg_v, sem1).start()
            pltpu.make_async_copy(rd_sh.at[dst_v], rg_v, sem1).wait()

            def inner(i, v):
                e16 = 16 * i + lane
                rv = plsc.load_gather(rg_v, [e16, lane * 0])
                at_v[pl.ds(16 * i, 16)] = s2_v[pl.ds(16 * i, 16)] * rv
                return v
            lax.fori_loop(0, CH // 16, inner, 0, unroll=4)

            @pl.when(base + CH <= etot)
            def _():
                pltpu.sync_copy(at_v, attn_h.at[pl.ds(base, CH)])

            if tail:
                @pl.when(base + CH - etot == CH - tail)
                def _():
                    pltpu.sync_copy(at_v.at[pl.ds(0, tail)],
                                    attn_h.at[pl.ds(etot - tail, tail)])
            return u
        lax.fori_loop(0, cpw, chunk, 0)

    return k(dst, s2, rdt)


def kernel(x, edge_index, W1, att_src1, att_dst1, b1,
           W2, att_src2, att_dst2, b2):
    n = x.shape[0]
    e = edge_index.shape[1]
    etot = e + n

    cpw = _cdiv(etot, NW * CH)
    ep = NW * cpw * CH
    np_ = BLK * _cdiv(n + 1, BLK)
    npt = np_ // NS
    zr = npt // 4

    loop = jnp.arange(n, dtype=jnp.int32)
    src = jnp.concatenate([edge_index[0].astype(jnp.int32), loop,
                           jnp.zeros((ep - etot,), jnp.int32)])
    dst = jnp.concatenate([edge_index[1].astype(jnp.int32), loop,
                           jnp.full((ep - etot,), n, jnp.int32)])
    x_p = jnp.zeros((np_, 2), jnp.float32).at[:n].set(x)
    x8_p = jnp.zeros((np_, 8), jnp.float32).at[:n, 0:2].set(x)
    z8 = jnp.zeros((zr, 8), jnp.float32)
    z16 = jnp.zeros((npt // 16, 16), jnp.float32)
    as1 = att_src1.reshape(64, 1)
    ad1 = att_dst1.reshape(64, 1)
    b1r = b1.reshape(1, 64)
    b2r = b2.reshape(1, 2)
    as2 = att_src2.reshape(1, 2)
    ad2 = att_dst2.reshape(1, 2)

    t1s, t1d = _tc1(x_p, W1, as1, ad1, np_, BLK)
    s1, den_p = _sca(src, dst, t1s, t1d, z8, np_, cpw)
    rden = _tc2(den_p, n, np_, BLK)
    attn1, g_p = _scb(src, dst, s1, rden, x8_p, z16, np_,
                      cpw * (CH // CHB), etot)
    t2 = _tc3(g_p, rden, W1, b1r, W2, as2, ad2, np_, BLK)
    s2, acc2_p = _l2a(src, dst, t2, z8, np_, cpw * (CH // CHB))
    h2_p, rdt = _tc4(acc2_p, b2r, n, np_, BLK)
    attn2 = _l2b(dst, s2, rdt, np_, cpw, etot)

    return (h2_p[:n],
            (attn1[:etot], attn2[:etot].reshape(etot, 1)))


# speedup vs baseline: 1.2137x; 1.0000x over previous
import functools

import jax
import jax.numpy as jnp
from jax import lax
from jax.experimental import pallas as pl
from jax.experimental.pallas import tpu as pltpu
from jax.experimental.pallas import tpu_sc as plsc

NC, NS, LANES = 2, 16, 16
NW = NC * NS
CH = 1024
CHB = 256
BLK = 2048


def _cdiv(a, b):
    return (a + b - 1) // b


def _mesh():
    return plsc.VectorSubcoreMesh(
        core_axis_name="c", subcore_axis_name="s",
        num_cores=NC, num_subcores=NS)


def _tc1(x_p, w1, as1, ad1, np_, blk):
    def body(x_ref, w1_ref, as_ref, ad_ref, ts_ref, td_ref):
        h = jnp.dot(x_ref[...], w1_ref[...],
                    preferred_element_type=jnp.float32)
        ri = lax.broadcasted_iota(jnp.int32, (64, 8), 0)
        ci = lax.broadcasted_iota(jnp.int32, (64, 8), 1)
        msk = (ci == (ri >> 3)).astype(jnp.float32)
        ts_ref[...] = jnp.dot(h, as_ref[...] * msk,
                              preferred_element_type=jnp.float32)
        td_ref[...] = jnp.dot(h, ad_ref[...] * msk,
                              preferred_element_type=jnp.float32)

    grid = (np_ // blk,)
    return pl.pallas_call(
        body,
        grid=grid,
        in_specs=[
            pl.BlockSpec((blk, 2), lambda i: (i, 0)),
            pl.BlockSpec((2, 64), lambda i: (0, 0)),
            pl.BlockSpec((64, 1), lambda i: (0, 0)),
            pl.BlockSpec((64, 1), lambda i: (0, 0)),
        ],
        out_specs=[
            pl.BlockSpec((blk, 8), lambda i: (i, 0)),
            pl.BlockSpec((blk, 8), lambda i: (i, 0)),
        ],
        out_shape=[
            jax.ShapeDtypeStruct((np_, 8), jnp.float32),
            jax.ShapeDtypeStruct((np_, 8), jnp.float32),
        ],
    )(x_p, w1, as1, ad1)


def _tc2(den_p, n, np_, blk):
    def body(dp_ref, rd_ref):
        i = pl.program_id(0)
        d = dp_ref[0] + dp_ref[1]
        rows = i * blk + lax.broadcasted_iota(jnp.int32, (blk, 8), 0)
        rd_ref[...] = jnp.where(rows < n, 1.0 / (d + 1e-16), 0.0)

    return pl.pallas_call(
        body,
        grid=(np_ // blk,),
        in_specs=[pl.BlockSpec((2, blk, 8), lambda i: (0, i, 0))],
        out_specs=pl.BlockSpec((blk, 8), lambda i: (i, 0)),
        out_shape=jax.ShapeDtypeStruct((np_, 8), jnp.float32),
    )(den_p)


def _tc3(g_p, rden, w1, b1r, w2, as2, ad2, np_, blk):
    def body(gp_ref, rd_ref, w1_ref, b1_ref, w2_ref, as2_ref, ad2_ref,
             t2_ref):
        g = gp_ref[0] + gp_ref[1]
        rd = rd_ref[...]
        gn0 = g[:, 0:8] * rd
        gn1 = g[:, 8:16] * rd
        ri = lax.broadcasted_iota(jnp.int32, (8, 64), 0)
        ci = lax.broadcasted_iota(jnp.int32, (8, 64), 1)
        e8 = (ri == (ci >> 3)).astype(jnp.float32)
        h1 = (jnp.dot(gn0, e8 * w1_ref[0:1, :],
                      preferred_element_type=jnp.float32)
              + jnp.dot(gn1, e8 * w1_ref[1:2, :],
                        preferred_element_type=jnp.float32)
              + b1_ref[...])
        h1 = jnp.where(h1 > 0, h1, jnp.exp(jnp.minimum(h1, 0.0)) - 1.0)
        h2p = jnp.dot(h1, w2_ref[...], preferred_element_type=jnp.float32)
        a2s = jnp.sum(h2p * as2_ref[...], axis=1, keepdims=True)
        a2d = jnp.sum(h2p * ad2_ref[...], axis=1, keepdims=True)
        t2_ref[...] = jnp.concatenate(
            [a2s, a2d, h2p, jnp.zeros_like(h2p), jnp.zeros_like(h2p)],
            axis=1)

    return pl.pallas_call(
        body,
        grid=(np_ // blk,),
        in_specs=[
            pl.BlockSpec((2, blk, 16), lambda i: (0, i, 0)),
            pl.BlockSpec((blk, 8), lambda i: (i, 0)),
            pl.BlockSpec((2, 64), lambda i: (0, 0)),
            pl.BlockSpec((1, 64), lambda i: (0, 0)),
            pl.BlockSpec((64, 2), lambda i: (0, 0)),
            pl.BlockSpec((1, 2), lambda i: (0, 0)),
            pl.BlockSpec((1, 2), lambda i: (0, 0)),
        ],
        out_specs=pl.BlockSpec((blk, 8), lambda i: (i, 0)),
        out_shape=jax.ShapeDtypeStruct((np_, 8), jnp.float32),
    )(g_p, rden, w1, b1r, w2, as2, ad2)


def _tc4(acc2_p, b2r, n, np_, blk):
    def body(ap_ref, b2_ref, h2_ref, rdt_ref):
        i = pl.program_id(0)
        a = ap_ref[0] + ap_ref[1]
        rows = i * blk + lax.broadcasted_iota(jnp.int32, (blk, 1), 0)
        rd2 = jnp.where(rows < n, 1.0 / (a[:, 0:1] + 1e-16), 0.0)
        h2_ref[...] = a[:, 1:3] * rd2 + b2_ref[...]
        rdt_ref[...] = jnp.broadcast_to(rd2, (blk, 8))

    return pl.pallas_call(
        body,
        grid=(np_ // blk,),
        in_specs=[
            pl.BlockSpec((2, blk, 8), lambda i: (0, i, 0)),
            pl.BlockSpec((1, 2), lambda i: (0, 0)),
        ],
        out_specs=[
            pl.BlockSpec((blk, 2), lambda i: (i, 0)),
            pl.BlockSpec((blk, 8), lambda i: (i, 0)),
        ],
        out_shape=[
            jax.ShapeDtypeStruct((np_, 2), jnp.float32),
            jax.ShapeDtypeStruct((np_, 8), jnp.float32),
        ],
    )(acc2_p, b2r)


def _sca(src, dst, t1s, t1d, z8, np_, cpw):
    ep = src.shape[0]
    npt = np_ // NS
    zr = z8.shape[0]
    nz = npt // zr

    @functools.partial(
        pl.kernel,
        out_type=(jax.ShapeDtypeStruct((ep, 8), jnp.float32),
                  jax.ShapeDtypeStruct((NC, np_, 8), jnp.float32)),
        mesh=_mesh(),
        compiler_params=pltpu.CompilerParams(needs_layout_passes=False, use_tc_tiling_on_sc=False),
        scratch_types=[
            pltpu.VMEM((CH,), jnp.int32), pltpu.VMEM((CH,), jnp.int32),
            pltpu.VMEM((CH, 8), jnp.float32), pltpu.VMEM((CH, 8), jnp.float32),
            pltpu.VMEM((CH, 8), jnp.float32), pltpu.VMEM((zr, 8), jnp.float32),
            pltpu.VMEM_SHARED((np_, 8), jnp.float32),
            pltpu.SemaphoreType.DMA, pltpu.SemaphoreType.DMA,
        ],
    )
    def k(src_h, dst_h, t1s_h, t1d_h, z8_h, s_out_h, den_out_h,
          src_v, dst_v, gs_v, gd_v, s_v, zb_v, den_sh, sem1, sem2):
        c = lax.axis_index("c")
        t = lax.axis_index("s")
        wid = t * NC + c
        lane = lax.iota(jnp.int32, 16)

        pltpu.sync_copy(z8_h, zb_v)

        def zloop(r, u):
            pltpu.sync_copy(zb_v, den_sh.at[pl.ds(t * npt + r * zr, zr), :])
            return u
        lax.fori_loop(0, nz, zloop, 0)
        plsc.subcore_barrier()

        def chunk(kk, u):
            base = (wid * cpw + kk) * CH
            pltpu.sync_copy(src_h.at[pl.ds(base, CH)], src_v)
            pltpu.sync_copy(dst_h.at[pl.ds(base, CH)], dst_v)
            cp1 = pltpu.make_async_copy(t1s_h.at[src_v], gs_v, sem1)
            cp1.start()
            cp2 = pltpu.make_async_copy(t1d_h.at[dst_v], gd_v, sem2)
            cp2.start()
            cp1.wait()
            cp2.wait()

            def inner(i, v):
                e16 = 2 * i + (lane >> 3)
                h16 = lane & 7
                a = (plsc.load_gather(gs_v, [e16, h16])
                     + plsc.load_gather(gd_v, [e16, h16]))
                sv = jnp.exp(jnp.maximum(a, 0.2 * a))
                plsc.store_scatter(s_v, [e16, h16], sv)
                return v
            lax.fori_loop(0, CH * 8 // 16, inner, 0, unroll=4)

            pltpu.sync_copy(s_v, s_out_h.at[pl.ds(base, CH), :])
            pltpu.sync_copy(s_v, den_sh.at[dst_v], add=True)
            return u
        lax.fori_loop(0, cpw, chunk, 0)
        plsc.subcore_barrier()

        def rloop(r, u):
            sl = pl.ds(t * npt + r * zr, zr)
            pltpu.sync_copy(den_sh.at[sl, :], zb_v)
            pltpu.sync_copy(zb_v, den_out_h.at[c, sl, :])
            return u
        lax.fori_loop(0, nz, rloop, 0)

    return k(src, dst, t1s, t1d, z8)


def _scb(src, dst, s_in, rden, x_p, z16, np_, cpw, etot):
    ep = src.shape[0]
    npt = np_ // NS
    zr = z16.shape[0]
    nz = npt // zr
    tail = etot % CHB

    @functools.partial(
        pl.kernel,
        out_type=(jax.ShapeDtypeStruct((etot * 8,), jnp.float32),
                  jax.ShapeDtypeStruct((NC, np_, 16), jnp.float32)),
        mesh=_mesh(),
        compiler_params=pltpu.CompilerParams(needs_layout_passes=False, use_tc_tiling_on_sc=False),
        scratch_types=[
            pltpu.VMEM((CHB,), jnp.int32), pltpu.VMEM((CHB,), jnp.int32),
            pltpu.VMEM((CHB, 8), jnp.float32), pltpu.VMEM((CHB, 8), jnp.float32),
            pltpu.VMEM((CHB * 8,), jnp.float32), pltpu.VMEM((CHB, 8), jnp.float32),
            pltpu.VMEM((CHB, 16), jnp.float32),
            pltpu.VMEM((zr, 16), jnp.float32),
            pltpu.VMEM_SHARED((np_, 16), jnp.float32),
            pltpu.SemaphoreType.DMA, pltpu.SemaphoreType.DMA,
        ],
    )
    def k(src_h, dst_h, s_in_h, rden_h, x_h, z16_h, attn_h, g_out_h,
          src_v, dst_v, s_v, rd_v, at_v, x_v, p_v, zb_v,
          g_sh, sem1, sem2):
        c = lax.axis_index("c")
        t = lax.axis_index("s")
        wid = t * NC + c
        lane = lax.iota(jnp.int32, 16)

        pltpu.sync_copy(z16_h, zb_v)

        def zloop(r, u):
            pltpu.sync_copy(zb_v, g_sh.at[pl.ds(t * npt + r * zr, zr), :])
            return u
        lax.fori_loop(0, nz, zloop, 0)
        plsc.subcore_barrier()

        def chunk(kk, u):
            base = (wid * cpw + kk) * CHB
            pltpu.sync_copy(src_h.at[pl.ds(base, CHB)], src_v)
            pltpu.sync_copy(dst_h.at[pl.ds(base, CHB)], dst_v)
            pltpu.sync_copy(s_in_h.at[pl.ds(base, CHB), :], s_v)
            cp1 = pltpu.make_async_copy(rden_h.at[dst_v], rd_v, sem1)
            cp1.start()
            cp2 = pltpu.make_async_copy(x_h.at[src_v], x_v, sem2)
            cp2.start()
            cp1.wait()
            cp2.wait()

            def inner1(i, v):
                e16 = 2 * i + (lane >> 3)
                h16 = lane & 7
                at = (plsc.load_gather(s_v, [e16, h16])
                      * plsc.load_gather(rd_v, [e16, h16]))
                at_v[pl.ds(16 * i, 16)] = at
                return v
            lax.fori_loop(0, CHB * 8 // 16, inner1, 0, unroll=4)

            def inner2(e, v):
                eb = lane * 0 + e
                p = (plsc.load_gather(s_v, [eb, lane & 7])
                     * plsc.load_gather(x_v, [eb, lane >> 3]))
                plsc.store_scatter(p_v, [eb, lane], p)
                return v
            lax.fori_loop(0, CHB, inner2, 0, unroll=4)

            @pl.when(base + CHB <= etot)
            def _():
                pltpu.sync_copy(at_v, attn_h.at[pl.ds(base * 8, CHB * 8)])

            if tail:
                @pl.when(base + CHB - etot == CHB - tail)
                def _():
                    pltpu.sync_copy(
                        at_v.at[pl.ds(0, tail * 8)],
                        attn_h.at[pl.ds((etot - tail) * 8, tail * 8)])
            pltpu.sync_copy(p_v, g_sh.at[dst_v], add=True)
            return u
        lax.fori_loop(0, cpw, chunk, 0)
        plsc.subcore_barrier()

        def rloop(r, u):
            sl = pl.ds(t * npt + r * zr, zr)
            pltpu.sync_copy(g_sh.at[sl, :], zb_v)
            pltpu.sync_copy(zb_v, g_out_h.at[c, sl, :])
            return u
        lax.fori_loop(0, nz, rloop, 0)

    return k(src, dst, s_in, rden, x_p, z16)


def _l2a(src, dst, t2, z8, np_, cpw):
    ep = src.shape[0]
    npt = np_ // NS
    zr = z8.shape[0]
    nz = npt // zr

    @functools.partial(
        pl.kernel,
        out_type=(jax.ShapeDtypeStruct((ep,), jnp.float32),
                  jax.ShapeDtypeStruct((NC, np_, 8), jnp.float32)),
        mesh=_mesh(),
        compiler_params=pltpu.CompilerParams(needs_layout_passes=False, use_tc_tiling_on_sc=False),
        scratch_types=[
            pltpu.VMEM((CHB,), jnp.int32), pltpu.VMEM((CHB,), jnp.int32),
            pltpu.VMEM((CHB, 8), jnp.float32), pltpu.VMEM((CHB, 8), jnp.float32),
            pltpu.VMEM((CHB,), jnp.float32), pltpu.VMEM((CHB, 8), jnp.float32),
            pltpu.VMEM((zr, 8), jnp.float32),
            pltpu.VMEM_SHARED((np_, 8), jnp.float32),
            pltpu.VMEM_SHARED((np_, 8), jnp.float32),
            pltpu.SemaphoreType.DMA, pltpu.SemaphoreType.DMA,
        ],
    )
    def k(src_h, dst_h, t2_h, z8_h, s2_out_h, acc_out_h,
          src_v, dst_v, g2s_v, g2d_v, s2_v, p2_v, zb_v,
          t2_sh, acc_sh, sem1, sem2):
        c = lax.axis_index("c")
        t = lax.axis_index("s")
        wid = t * NC + c
        lane = lax.iota(jnp.int32, 16)

        pltpu.sync_copy(z8_h, zb_v)

        def zloop(r, u):
            pltpu.sync_copy(zb_v, acc_sh.at[pl.ds(t * npt + r * zr, zr), :])
            return u
        lax.fori_loop(0, nz, zloop, 0)

        def sloop(r, u):
            sl = pl.ds(t * npt + r * zr, zr)
            pltpu.sync_copy(t2_h.at[sl, :], zb_v)
            pltpu.sync_copy(zb_v, t2_sh.at[sl, :])
            return u
        lax.fori_loop(0, nz, sloop, 0)
        plsc.subcore_barrier()

        def chunk(kk, u):
            base = (wid * cpw + kk) * CHB
            pltpu.sync_copy(src_h.at[pl.ds(base, CHB)], src_v)
            pltpu.sync_copy(dst_h.at[pl.ds(base, CHB)], dst_v)
            cp1 = pltpu.make_async_copy(t2_sh.at[src_v], g2s_v, sem1)
            cp1.start()
            cp2 = pltpu.make_async_copy(t2_sh.at[dst_v], g2d_v, sem2)
            cp2.start()
            cp1.wait()
            cp2.wait()

            def inner1(i, v):
                e16 = 16 * i + lane
                a = (plsc.load_gather(g2s_v, [e16, lane * 0])
                     + plsc.load_gather(g2d_v, [e16, lane * 0 + 1]))
                s2 = jnp.exp(jnp.maximum(a, 0.2 * a))
                s2_v[pl.ds(16 * i, 16)] = s2
                return v
            lax.fori_loop(0, CHB // 16, inner1, 0, unroll=4)

            def inner2(i, v):
                er = 2 * i + (lane >> 3)
                col = lane & 7
                s2g = plsc.load_gather(s2_v, [er])
                hv = plsc.load_gather(g2s_v, [er, jnp.minimum(col + 1, 7)])
                p2 = jnp.where(col == 0, s2g, s2g * hv)
                plsc.store_scatter(p2_v, [er, col], p2)
                return v
            lax.fori_loop(0, CHB // 2, inner2, 0, unroll=4)

            pltpu.sync_copy(s2_v, s2_out_h.at[pl.ds(base, CHB)])
            pltpu.sync_copy(p2_v, acc_sh.at[dst_v], add=True)
            return u
        lax.fori_loop(0, cpw, chunk, 0)
        plsc.subcore_barrier()

        def rloop(r, u):
            sl = pl.ds(t * npt + r * zr, zr)
            pltpu.sync_copy(acc_sh.at[sl, :], zb_v)
            pltpu.sync_copy(zb_v, acc_out_h.at[c, sl, :])
            return u
        lax.fori_loop(0, nz, rloop, 0)

    return k(src, dst, t2, z8)


def _l2b(dst, s2, rdt, np_, cpw, etot):
    ep = dst.shape[0]
    npt = np_ // NS
    zr = npt // 4
    nz = 4
    tail = etot % CH

    @functools.partial(
        pl.kernel,
        out_type=jax.ShapeDtypeStruct((etot,), jnp.float32),
        mesh=_mesh(),
        compiler_params=pltpu.CompilerParams(needs_layout_passes=False, use_tc_tiling_on_sc=False),
        scratch_types=[
            pltpu.VMEM((CH,), jnp.int32), pltpu.VMEM((CH,), jnp.float32),
            pltpu.VMEM((CH, 8), jnp.float32), pltpu.VMEM((CH,), jnp.float32),
            pltpu.VMEM((zr, 8), jnp.float32),
            pltpu.VMEM_SHARED((np_, 8), jnp.float32),
            pltpu.SemaphoreType.DMA,
        ],
    )
    def k(dst_h, s2_h, rdt_h, attn_h,
          dst_v, s2_v, rg_v, at_v, zb_v, rd_sh, sem1):
        c = lax.axis_index("c")
        t = lax.axis_index("s")
        wid = t * NC + c
        lane = lax.iota(jnp.int32, 16)

        def sloop(r, u):
            sl = pl.ds(t * npt + r * zr, zr)
            pltpu.sync_copy(rdt_h.at[sl, :], zb_v)
            pltpu.sync_copy(zb_v, rd_sh.at[sl, :])
            return u
        lax.fori_loop(0, nz, sloop, 0)
        plsc.subcore_barrier()

        def chunk(kk, u):
            base = (wid * cpw + kk) * CH
            pltpu.sync_copy(dst_h.at[pl.ds(base, CH)], dst_v)
            pltpu.sync_copy(s2_h.at[pl.ds(base, CH)], s2_v)
            pltpu.make_async_copy(rd_sh.at[dst_v], rg_v, sem1).start()
            pltpu.make_async_copy(rd_sh.at[dst_v], rg_v, sem1).wait()

            def inner(i, v):
                e16 = 16 * i + lane
                rv = plsc.load_gather(rg_v, [e16, lane * 0])
                at_v[pl.ds(16 * i, 16)] = s2_v[pl.ds(16 * i, 16)] * rv
                return v
            lax.fori_loop(0, CH // 16, inner, 0, unroll=4)

            @pl.when(base + CH <= etot)
            def _():
                pltpu.sync_copy(at_v, attn_h.at[pl.ds(base, CH)])

            if tail:
                @pl.when(base + CH - etot == CH - tail)
                def _():
                    pltpu.sync_copy(at_v.at[pl.ds(0, tail)],
                                    attn_h.at[pl.ds(etot - tail, tail)])
            return u
        lax.fori_loop(0, cpw, chunk, 0)

    return k(dst, s2, rdt)


def kernel(x, edge_index, W1, att_src1, att_dst1, b1,
           W2, att_src2, att_dst2, b2):
    n = x.shape[0]
    e = edge_index.shape[1]
    etot = e + n

    cpw = _cdiv(etot, NW * CH)
    ep = NW * cpw * CH
    np_ = BLK * _cdiv(n + 1, BLK)
    npt = np_ // NS
    zr = npt // 4

    loop = jnp.arange(n, dtype=jnp.int32)
    src = jnp.concatenate([edge_index[0].astype(jnp.int32), loop,
                           jnp.zeros((ep - etot,), jnp.int32)])
    dst = jnp.concatenate([edge_index[1].astype(jnp.int32), loop,
                           jnp.full((ep - etot,), n, jnp.int32)])
    x_p = jnp.zeros((np_, 2), jnp.float32).at[:n].set(x)
    x8_p = jnp.zeros((np_, 8), jnp.float32).at[:n, 0:2].set(x)
    z8 = jnp.zeros((zr, 8), jnp.float32)
    z16 = jnp.zeros((npt // 16, 16), jnp.float32)
    as1 = att_src1.reshape(64, 1)
    ad1 = att_dst1.reshape(64, 1)
    b1r = b1.reshape(1, 64)
    b2r = b2.reshape(1, 2)
    as2 = att_src2.reshape(1, 2)
    ad2 = att_dst2.reshape(1, 2)

    t1s, t1d = _tc1(x_p, W1, as1, ad1, np_, BLK)
    s1, den_p = _sca(src, dst, t1s, t1d, z8, np_, cpw)
    rden = _tc2(den_p, n, np_, BLK)
    attn1f, g_p = _scb(src, dst, s1, rden, x8_p, z16, np_,
                       cpw * (CH // CHB), etot)
    t2 = _tc3(g_p, rden, W1, b1r, W2, as2, ad2, np_, BLK)
    s2, acc2_p = _l2a(src, dst, t2, z8, np_, cpw * (CH // CHB))
    h2_p, rdt = _tc4(acc2_p, b2r, n, np_, BLK)
    attn2f = _l2b(dst, s2, rdt, np_, cpw, etot)

    return (h2_p[:n],
            (attn1f.reshape(etot, 8), attn2f.reshape(etot, 1)))
